# baseline (speedup 1.0000x reference)
import functools

import jax
import jax.numpy as jnp
from jax import lax
from jax.experimental import pallas as pl
from jax.experimental.pallas import tpu as pltpu
from jax.experimental.pallas import tpu_sc as plsc

D = 64
L = 16
NC = 2
NS = 16
NW = NC * NS

SUB = 128
SUBS_PER_CHUNK = 4
CHUNK = SUB * SUBS_PER_CHUNK
GROUPS = CHUNK // L


@jax.jit
def _sc_gather(idx2d, table):
    n_rows, _ = idx2d.shape
    n = n_rows * SUB
    per_w = n // NW
    n_chunks = per_w // CHUNK
    mesh = plsc.VectorSubcoreMesh(core_axis_name="c", subcore_axis_name="s")

    @functools.partial(
        pl.kernel,
        out_type=jax.ShapeDtypeStruct((n, D), jnp.float32),
        mesh=mesh,
        compiler_params=pltpu.CompilerParams(
            needs_layout_passes=False, use_tc_tiling_on_sc=False),
        scratch_types=[
            pltpu.VMEM((SUBS_PER_CHUNK, SUB), jnp.int32),
            pltpu.VMEM((CHUNK, D), jnp.float32),
            pltpu.SemaphoreType.DMA,
        ],
    )
    def k(idx_hbm, tab_hbm, out_hbm, idx_v, rows_v, gsem):
        wid = lax.axis_index("s") * NC + lax.axis_index("c")
        chunk0 = wid * n_chunks

        def chunk_body(g, carry):
            c = chunk0 + g
            pltpu.sync_copy(
                idx_hbm.at[pl.ds(c * SUBS_PER_CHUNK, SUBS_PER_CHUNK)], idx_v)
            handles = [
                pltpu.async_copy(
                    tab_hbm.at[idx_v.at[j]],
                    rows_v.at[pl.ds(j * SUB, SUB)],
                    gsem,
                )
                for j in range(SUBS_PER_CHUNK)
            ]
            for h in handles:
                h.wait()

            def grp_body(i, carry2):
                j = i // (SUB // L)
                off = (i % (SUB // L)) * L
                idx16 = idx_v[j, pl.ds(off, L)]
                npad = plsc.all_reduce_population_count(idx16 == 0)

                @pl.when(npad[0] > 0)
                def _fix():
                    for r in range(L):
                        x = idx16[r]

                        @pl.when(x == 0)
                        def _zero():
                            row = i * L + r
                            zero = jnp.zeros((L,), jnp.float32)
                            for cc in range(D // L):
                                rows_v[row, pl.ds(cc * L, L)] = zero

                return carry2

            lax.fori_loop(0, GROUPS, grp_body, 0)

            pltpu.sync_copy(rows_v, out_hbm.at[pl.ds(c * CHUNK, CHUNK)])
            return carry

        lax.fori_loop(0, n_chunks, chunk_body, 0)

    return k(idx2d, table)


def kernel(event, table):
    b, h = event.shape
    idx2d = event.reshape(b * h // SUB, SUB)
    out = _sc_gather(idx2d, table)
    return out.reshape(b, h, D)


# speedup vs baseline: 1.1165x; 1.1165x over previous
import functools

import jax
import jax.numpy as jnp
from jax import lax
from jax.experimental import pallas as pl
from jax.experimental.pallas import tpu as pltpu
from jax.experimental.pallas import tpu_sc as plsc

D = 64
L = 16
NC = 2
NS = 16
NW = NC * NS

SUB = 128
SUBS_PER_CHUNK = 4
CHUNK = SUB * SUBS_PER_CHUNK
GROUPS = CHUNK // L
NB = 3


@jax.jit
def _sc_gather(idx2d, table):
    n_rows, _ = idx2d.shape
    n = n_rows * SUB
    per_w = n // NW
    n_chunks = per_w // CHUNK
    mesh = plsc.VectorSubcoreMesh(core_axis_name="c", subcore_axis_name="s")

    @functools.partial(
        pl.kernel,
        out_type=jax.ShapeDtypeStruct((n, D), jnp.float32),
        mesh=mesh,
        compiler_params=pltpu.CompilerParams(
            needs_layout_passes=False, use_tc_tiling_on_sc=False),
        scratch_types=[
            pltpu.VMEM((NB, SUBS_PER_CHUNK, SUB), jnp.int32),
            pltpu.VMEM((NB, CHUNK, D), jnp.float32),
            pltpu.SemaphoreType.DMA((NB,)),
            pltpu.SemaphoreType.DMA((NB,)),
            pltpu.SemaphoreType.DMA((NB,)),
        ],
    )
    def k(idx_hbm, tab_hbm, out_hbm, idx_v, rows_v, isem, gsem, osem):
        wid = lax.axis_index("s") * NC + lax.axis_index("c")
        chunk0 = wid * n_chunks

        def idx_hslice(c):
            return idx_hbm.at[pl.ds((chunk0 + c) * SUBS_PER_CHUNK,
                                    SUBS_PER_CHUNK)]

        def out_hslice(c):
            return out_hbm.at[pl.ds((chunk0 + c) * CHUNK, CHUNK)]

        def start_idx(c, b):
            pltpu.async_copy(idx_hslice(c), idx_v.at[b], isem.at[b])

        def wait_idx(c, b):
            pltpu.make_async_copy(idx_hslice(c), idx_v.at[b],
                                  isem.at[b]).wait()

        def start_gather(b):
            for j in range(SUBS_PER_CHUNK):
                pltpu.async_copy(
                    tab_hbm.at[idx_v.at[b].at[j]],
                    rows_v.at[b].at[pl.ds(j * SUB, SUB)],
                    gsem.at[b],
                )

        def wait_gather(b):
            for j in range(SUBS_PER_CHUNK):
                pltpu.make_async_copy(
                    tab_hbm.at[idx_v.at[b].at[j]],
                    rows_v.at[b].at[pl.ds(j * SUB, SUB)],
                    gsem.at[b],
                ).wait()

        def start_out(c, b):
            pltpu.async_copy(rows_v.at[b], out_hslice(c), osem.at[b])

        def wait_out(c, b):
            pltpu.make_async_copy(rows_v.at[b], out_hslice(c),
                                  osem.at[b]).wait()

        for b in range(NB):
            start_idx(b, b)
        wait_idx(0, 0)
        start_gather(0)

        def chunk_body(g, carry):
            b = lax.rem(g, NB)

            @pl.when(g + 1 < n_chunks)
            def _next_gather():
                b1 = lax.rem(g + 1, NB)

                @pl.when(g + 1 >= NB)
                def _reuse():
                    wait_out(g + 1 - NB, b1)

                wait_idx(g + 1, b1)
                start_gather(b1)

            wait_gather(b)

            def grp_body(i, carry2):
                j = i // (SUB // L)
                off = (i % (SUB // L)) * L
                idx16 = idx_v[b, j, pl.ds(off, L)]
                npad = plsc.all_reduce_population_count(idx16 == 0)

                @pl.when(npad[0] > 0)
                def _fix():
                    for r in range(L):
                        @pl.when(idx16[r] == 0)
                        def _zero():
                            row = i * L + r
                            zero = jnp.zeros((L,), jnp.float32)
                            for cc in range(D // L):
                                rows_v[b, row, pl.ds(cc * L, L)] = zero

                return carry2

            lax.fori_loop(0, GROUPS, grp_body, 0)

            start_out(g, b)

            @pl.when(g + NB < n_chunks)
            def _next_idx():
                start_idx(g + NB, b)

            return carry

        lax.fori_loop(0, n_chunks, chunk_body, 0)

        for c in range(n_chunks - NB, n_chunks):
            wait_out(c, c % NB)

    return k(idx2d, table)


def kernel(event, table):
    b, h = event.shape
    idx2d = event.reshape(b * h // SUB, SUB)
    out = _sc_gather(idx2d, table)
    return out.reshape(b, h, D)
